# Initial kernel scaffold; baseline (speedup 1.0000x reference)
import jax
import jax.numpy as jnp
from jax.experimental import pallas as pl


def kernel(vertices, feature_map):
    raise NotImplementedError("write your pallas kernel here")


# speedup vs baseline: 2305.3032x; 2305.3032x over previous
import functools

import jax
import jax.numpy as jnp
from jax import lax
from jax.experimental import pallas as pl
from jax.experimental.pallas import tpu as pltpu
from jax.experimental.pallas import tpu_sc as plsc

_POOLING_RATE = 4
_K = 16


_BR = 256


def _knn_body(vs_ref, vt_ref, out_ref):
    vsb = vs_ref[0]
    vt = vt_ref[0]
    n = vt.shape[1]
    inner = lax.dot_general(vsb, vt, (((1,), (0,)), ((), ())),
                            preferred_element_type=jnp.float32)
    q = jnp.sum(vt * vt, axis=0)
    qs = jnp.sum(vsb * vsb, axis=1)
    d = (-2.0 * inner + q[None, :]) + qs[:, None]
    cols = lax.broadcasted_iota(jnp.int32, d.shape, 1)
    off = pl.program_id(0) * n
    big = jnp.float32(1e30)
    for k in range(_K + 1):
        m = jnp.min(d, axis=1)
        eq = d == m[:, None]
        idx = jnp.min(jnp.where(eq, cols, n), axis=1)
        d = jnp.where(eq, big, d)
        if k > 0:
            out_ref[0, k - 1, :] = idx + off


def _knn_indices(vs, vt):
    bs, p, _ = vs.shape
    n = vt.shape[2]
    return pl.pallas_call(
        _knn_body,
        grid=(bs, p // _BR),
        in_specs=[
            pl.BlockSpec((1, _BR, 3), lambda b, r: (b, r, 0)),
            pl.BlockSpec((1, 3, n), lambda b, r: (b, 0, 0)),
        ],
        out_specs=pl.BlockSpec((1, _K, _BR), lambda b, r: (b, 0, r)),
        out_shape=jax.ShapeDtypeStruct((bs, _K, p), jnp.int32),
    )(vs, vt)


_CP = 8


def _make_pool_kernel(rows_total, d, pw, nbatch, p):
    info = plsc.get_sparse_core_info()
    nc, ns = info.num_cores, info.num_subcores
    steps = pw // _CP
    wpb = p // pw

    mesh = plsc.VectorSubcoreMesh(core_axis_name="c", subcore_axis_name="s")

    @functools.partial(
        pl.kernel,
        mesh=mesh,
        out_type=jax.ShapeDtypeStruct((rows_total, d), jnp.float32),
        scratch_types=[
            pltpu.VMEM((_K, pw), jnp.int32),
            pltpu.VMEM((_K, _CP, d), jnp.float32),
            pltpu.VMEM((pw, d), jnp.float32),
            pltpu.SemaphoreType.DMA,
        ],
        compiler_params=pltpu.CompilerParams(use_tc_tiling_on_sc=False),
    )
    def pool_kernel(fm_hbm, nbr_hbm, out_hbm, idx_v, gbuf, obuf, sem):
        wid = lax.axis_index("s") * nc + lax.axis_index("c")
        b = wid // wpb
        pbase = (wid % wpb) * pw
        gbase = wid * pw
        pltpu.sync_copy(nbr_hbm.at[b, :, pl.ds(pbase, pw)], idx_v)

        def step(s, carry):
            handles = []
            for k in range(_K):
                handles.append(pltpu.async_copy(
                    fm_hbm.at[idx_v.at[k, pl.ds(s * _CP, _CP)]],
                    gbuf.at[k], sem))
            for h in handles:
                h.wait()
            for pp in range(_CP):
                row = s * _CP + pp
                for j in range(d // 16):
                    sl = pl.ds(j * 16, 16)
                    acc = gbuf[0, pp, sl]
                    for k in range(1, _K):
                        acc = jnp.maximum(acc, gbuf[k, pp, sl])
                    obuf[row, sl] = acc
            return carry

        lax.fori_loop(0, steps, step, 0)
        pltpu.sync_copy(obuf, out_hbm.at[pl.ds(gbase, pw)])

    return pool_kernel


def kernel(vertices, feature_map):
    bs, n, _ = vertices.shape
    c = feature_map.shape[1]
    r = feature_map.shape[-1]
    p = n // _POOLING_RATE
    d = c * r

    sample_idx = jax.random.permutation(jax.random.key(42), n)[:p]
    sample_idx = sample_idx.astype(jnp.int32)

    vt = jnp.transpose(vertices, (0, 2, 1))
    vs = jnp.take(vertices, sample_idx, axis=1)
    nbr = _knn_indices(vs, vt)

    fm_t = jnp.transpose(feature_map, (0, 2, 1, 3)).reshape(bs * n, d)

    nw = 32
    pw = (bs * p) // nw
    pool = _make_pool_kernel(bs * p, d, pw, bs, p)
    pooled = pool(fm_t, nbr)

    feature_map_pool = pooled.reshape(bs, p, c, r).transpose(0, 2, 1, 3)
    return (vs, feature_map_pool)
